# Initial kernel scaffold; baseline (speedup 1.0000x reference)
import jax
import jax.numpy as jnp
from jax.experimental import pallas as pl


def kernel(graph_emd, cluster_id, prompts):
    raise NotImplementedError("write your pallas kernel here")


# speedup vs baseline: 1.1760x; 1.1760x over previous
import functools

import jax
import jax.numpy as jnp
from jax import lax
from jax.experimental import pallas as pl
from jax.experimental.pallas import tpu as pltpu
from jax.experimental.pallas import tpu_sc as plsc

B = 4096
C = 1000
V = 50
D = 128
ROW = V * D
VPAD = 64
NW = 32
EPW = B // NW
CH = 8
NCHUNK = EPW // CH
NK = D // 16


def _rsqrt16(x):
    i = plsc.bitcast(x, jnp.int32)
    i = jnp.int32(0x5F3759DF) - lax.shift_right_arithmetic(i, 1)
    y = plsc.bitcast(i, jnp.float32)
    for _ in range(3):
        y = y * (jnp.float32(1.5) - jnp.float32(0.5) * x * y * y)
    return y


def _sc_body(gemd, cid, ptab, out, idx_v, b_v, rows_v, dbuf, nbuf, tbuf, obuf,
             sem):
    wid = lax.axis_index("s") * 2 + lax.axis_index("c")
    base = wid * EPW
    iota = lax.iota(jnp.int32, 16)
    xor_masks = [iota ^ m for m in (8, 4, 2, 1)]

    def chunk_body(c, carry):
        eb = base + c * CH
        pltpu.sync_copy(cid.at[pl.ds(eb, CH)], idx_v)
        pltpu.sync_copy(gemd.at[pl.ds(eb, CH)], b_v)
        pltpu.async_copy(ptab.at[idx_v], rows_v, sem).wait()

        def elem_body(e, ecarry):
            bks = [b_v[e, pl.ds(k * 16, 16)] for k in range(NK)]
            accnb = bks[0] * bks[0]
            for k in range(1, NK):
                accnb = accnb + bks[k] * bks[k]
            for m in xor_masks:
                tbuf[pl.ds(0, 16)] = accnb
                accnb = accnb + plsc.load_gather(tbuf, [m])
            rnb = _rsqrt16(jnp.maximum(accnb, jnp.float32(1e-16)))
            for g in range(4):
                nj = 16 if g < 3 else V - 48
                for j in range(nj):
                    v = g * 16 + j
                    av = rows_v[e, pl.ds(v * D, 16)]
                    accd = av * bks[0]
                    accn = av * av
                    for k in range(1, NK):
                        av = rows_v[e, pl.ds(v * D + k * 16, 16)]
                        accd = accd + av * bks[k]
                        accn = accn + av * av
                    dbuf[pl.ds(j * 16, 16)] = accd
                    nbuf[pl.ds(j * 16, 16)] = accn
                gidx = iota * 16
                dsum = plsc.load_gather(dbuf, [gidx])
                nsum = plsc.load_gather(nbuf, [gidx])
                for j in range(1, 16):
                    gj = gidx + j
                    dsum = dsum + plsc.load_gather(dbuf, [gj])
                    nsum = nsum + plsc.load_gather(nbuf, [gj])
                rna = _rsqrt16(jnp.maximum(nsum, jnp.float32(1e-16)))
                obuf[e, pl.ds(g * 16, 16)] = dsum * rna * rnb
            return ecarry

        lax.fori_loop(0, CH, elem_body, 0)
        pltpu.sync_copy(obuf, out.at[pl.ds(eb, CH)])
        return carry

    lax.fori_loop(0, NCHUNK, chunk_body, 0)


@jax.jit
def _sc_cosine(gemd, cid, ptab):
    mesh = plsc.VectorSubcoreMesh(core_axis_name="c", subcore_axis_name="s")
    run = functools.partial(
        pl.kernel,
        mesh=mesh,
        out_type=jax.ShapeDtypeStruct((B, VPAD), jnp.float32),
        compiler_params=pltpu.CompilerParams(needs_layout_passes=False),
        scratch_types=[
            pltpu.VMEM((CH,), jnp.int32),
            pltpu.VMEM((CH, D), jnp.float32),
            pltpu.VMEM((CH, ROW), jnp.float32),
            pltpu.VMEM((256,), jnp.float32),
            pltpu.VMEM((256,), jnp.float32),
            pltpu.VMEM((16,), jnp.float32),
            pltpu.VMEM((CH, VPAD), jnp.float32),
            pltpu.SemaphoreType.DMA,
        ],
    )(_sc_body)
    return run(gemd, cid, ptab)


def kernel(graph_emd, cluster_id, prompts):
    cid = cluster_id.astype(jnp.int32)
    ptab = prompts.reshape(C, ROW)
    out = _sc_cosine(graph_emd, cid, ptab)
    return out[:, :V].reshape(B, 10, 5)
